# Initial kernel scaffold; baseline (speedup 1.0000x reference)
import functools

import jax
import jax.numpy as jnp
from jax import lax
from jax.experimental import pallas as pl
from jax.experimental.pallas import tpu as pltpu
from jax.experimental.pallas import tpu_sc as plsc

SERIES = 8192
B = 4096
L = 200
NC, NS, LN = 2, 16, 16
NW = NC * NS
RW = B // NW
G = 4
GW = G * SERIES
NG = RW // G
NB = 2
NV = (G * L) // LN

_mesh = plsc.VectorSubcoreMesh(core_axis_name="c", subcore_axis_name="s")


@functools.partial(
    pl.kernel,
    out_type=jax.ShapeDtypeStruct((B * SERIES,), jnp.float32),
    mesh=_mesh,
    scratch_types=[
        pltpu.VMEM((RW * L,), jnp.int32),
        pltpu.VMEM((NB * GW,), jnp.float32),
        pltpu.SemaphoreType.DMA,
        pltpu.SemaphoreType.DMA,
    ],
)
def _sc_build(x_hbm, out_hbm, idx_v, rows_v, sem0, sem1):
    wid = lax.axis_index("s") * NC + lax.axis_index("c")
    ibase = wid * (RW * L)
    obase = wid * (RW * SERIES)
    sems = (sem0, sem1)

    pltpu.sync_copy(x_hbm.at[pl.ds(ibase, RW * L)], idx_v)

    zero16 = jnp.zeros((LN,), jnp.float32)

    def zbody(j, _):
        rows_v[pl.ds(j * LN, LN)] = zero16
        return 0

    lax.fori_loop(0, (NB * GW) // LN, zbody, 0)

    iota = lax.iota(jnp.int32, LN)

    def scat(g, boff, build):
        def sbody(i, _):
            idx = idx_v[pl.ds(g * (G * L) + i * LN, LN)]
            r = (iota + i * LN) // L
            flat = boff + r * SERIES + idx
            if build:
                val = jnp.where(idx == 0, 0.0, 1.0).astype(jnp.float32)
            else:
                val = zero16
            plsc.store_scatter(rows_v, [flat], val)
            return 0

        lax.fori_loop(0, NV, sbody, 0)

    def dma_start(b, g):
        pltpu.async_copy(
            rows_v.at[pl.ds(b * GW, GW)],
            out_hbm.at[pl.ds(obase + g * GW, GW)],
            sems[b],
        )

    def dma_wait(b):
        pltpu.make_async_copy(
            rows_v.at[pl.ds(b * GW, GW)],
            out_hbm.at[pl.ds(obase, GW)],
            sems[b],
        ).wait()

    for b in range(NB):
        scat(jnp.int32(b), b * GW, True)
        dma_start(b, jnp.int32(b))

    def outer(t, _):
        for b in range(NB):
            g = t * NB + b
            dma_wait(b)
            scat(g - NB, b * GW, False)
            scat(g, b * GW, True)
            dma_start(b, g)
        return 0

    lax.fori_loop(1, NG // NB, outer, 0)

    for b in range(NB):
        dma_wait(b)


def kernel(x):
    x2 = x.reshape(B * L)
    flat = _sc_build(x2)
    return flat.reshape(B, SERIES)


# speedup vs baseline: 18.2043x; 18.2043x over previous
import functools

import jax
import jax.numpy as jnp
from jax import lax
from jax.experimental import pallas as pl
from jax.experimental.pallas import tpu as pltpu
from jax.experimental.pallas import tpu_sc as plsc

SERIES = 8192
B = 4096
L = 200
NC, NS, LN = 2, 16, 16
NW = NC * NS
RW = B // NW
G = 4
GW = G * SERIES
NG = RW // G
NB = 2
STARTS = tuple(c * LN for c in range(12)) + (L - LN,)

_mesh = plsc.VectorSubcoreMesh(core_axis_name="c", subcore_axis_name="s")


@functools.partial(
    pl.kernel,
    out_type=jax.ShapeDtypeStruct((B * SERIES,), jnp.float32),
    mesh=_mesh,
    scratch_types=[
        pltpu.VMEM((RW, L), jnp.int32),
        pltpu.VMEM((NB * GW,), jnp.float32),
        pltpu.SemaphoreType.DMA,
        pltpu.SemaphoreType.DMA,
    ],
    compiler_params=pltpu.CompilerParams(needs_layout_passes=False),
)
def _sc_build(x_hbm, out_hbm, idx_v, rows_v, sem0, sem1):
    wid = lax.axis_index("s") * NC + lax.axis_index("c")
    obase = wid * (RW * SERIES)
    sems = (sem0, sem1)

    pltpu.sync_copy(x_hbm.at[pl.ds(wid * RW, RW)], idx_v)

    zero16 = jnp.zeros((LN,), jnp.float32)

    def zbody(j, _):
        for u in range(4):
            rows_v[pl.ds((j * 4 + u) * LN, LN)] = zero16
        return 0

    lax.fori_loop(0, (NB * GW) // (4 * LN), zbody, 0)

    one16 = jnp.full((LN,), 1.0, jnp.float32)
    zero16i = jnp.zeros((LN,), jnp.int32)

    def scat(g, boff, build):
        for r in range(G):
            row = g * G + r
            basev = jnp.full((LN,), boff + r * SERIES, jnp.int32)
            for start in STARTS:
                idx = idx_v[row, pl.ds(start, LN)]
                flat = basev + idx
                if build:
                    val = jnp.where(idx == zero16i, zero16, one16)
                else:
                    val = zero16
                plsc.store_scatter(rows_v, [flat], val)

    def dma_start(b, g):
        pltpu.async_copy(
            rows_v.at[pl.ds(b * GW, GW)],
            out_hbm.at[pl.ds(obase + g * GW, GW)],
            sems[b],
        )

    def dma_wait(b):
        pltpu.make_async_copy(
            rows_v.at[pl.ds(b * GW, GW)],
            out_hbm.at[pl.ds(obase, GW)],
            sems[b],
        ).wait()

    for b in range(NB):
        scat(jnp.int32(b), b * GW, True)
        dma_start(b, jnp.int32(b))

    def outer(t, _):
        for b in range(NB):
            g = t * NB + b
            dma_wait(b)
            scat(g - NB, b * GW, False)
            scat(g, b * GW, True)
            dma_start(b, g)
        return 0

    lax.fori_loop(1, NG // NB, outer, 0)

    for b in range(NB):
        dma_wait(b)


def kernel(x):
    x2 = x.reshape(B, L)
    flat = _sc_build(x2)
    return flat.reshape(B, SERIES)


# speedup vs baseline: 52.1255x; 2.8634x over previous
import functools

import jax
import jax.numpy as jnp
from jax import lax
from jax.experimental import pallas as pl
from jax.experimental.pallas import tpu as pltpu
from jax.experimental.pallas import tpu_sc as plsc

SERIES = 8192
B = 4096
L = 200
NC, NS, LN = 2, 16, 16
NW = NC * NS
RW = B // NW
G = 4
GW = G * SERIES
NG = RW // G
NB = 2
STARTS = tuple(c * LN for c in range(12)) + (L - LN,)

_mesh = plsc.VectorSubcoreMesh(core_axis_name="c", subcore_axis_name="s")


@functools.partial(
    pl.kernel,
    out_type=jax.ShapeDtypeStruct((B, SERIES), jnp.float32),
    mesh=_mesh,
    scratch_types=[
        pltpu.VMEM((RW, L), jnp.int32),
        pltpu.VMEM((NB * G, SERIES), jnp.float32),
        pltpu.SemaphoreType.DMA,
        pltpu.SemaphoreType.DMA,
    ],
    compiler_params=pltpu.CompilerParams(needs_layout_passes=False),
)
def _sc_build(x_hbm, out_hbm, idx_v, rows_v, sem0, sem1):
    wid = lax.axis_index("s") * NC + lax.axis_index("c")
    orow = wid * RW
    sems = (sem0, sem1)

    pltpu.sync_copy(x_hbm.at[pl.ds(wid * RW, RW)], idx_v)

    zero16 = jnp.zeros((LN,), jnp.float32)

    def zbody(p, _):
        for j in range(NB * G):
            rows_v[j, pl.ds(p * LN, LN)] = zero16
        return 0

    lax.fori_loop(0, SERIES // LN, zbody, 0)

    one16 = jnp.full((LN,), 1.0, jnp.float32)
    zero16i = jnp.zeros((LN,), jnp.int32)

    def scat(g, brow, build):
        for r in range(G):
            row = g * G + r
            rvec = jnp.full((LN,), brow + r, jnp.int32)
            for start in STARTS:
                idx = idx_v[row, pl.ds(start, LN)]
                if build:
                    val = jnp.where(idx == zero16i, zero16, one16)
                else:
                    val = zero16
                plsc.store_scatter(rows_v, [rvec, idx], val)

    def dma_start(b, g):
        pltpu.async_copy(
            rows_v.at[pl.ds(b * G, G)],
            out_hbm.at[pl.ds(orow + g * G, G)],
            sems[b],
        )

    def dma_wait(b):
        pltpu.make_async_copy(
            rows_v.at[pl.ds(b * G, G)],
            out_hbm.at[pl.ds(orow, G)],
            sems[b],
        ).wait()

    for b in range(NB):
        scat(jnp.int32(b), b * G, True)
        dma_start(b, jnp.int32(b))

    def outer(t, _):
        for b in range(NB):
            g = t * NB + b
            dma_wait(b)
            scat(g - NB, b * G, False)
            scat(g, b * G, True)
            dma_start(b, g)
        return 0

    lax.fori_loop(1, NG // NB, outer, 0)

    for b in range(NB):
        dma_wait(b)


def kernel(x):
    return _sc_build(x.reshape(B, L))


# speedup vs baseline: 54.6147x; 1.0478x over previous
import functools

import jax
import jax.numpy as jnp
from jax import lax
from jax.experimental import pallas as pl
from jax.experimental.pallas import tpu as pltpu
from jax.experimental.pallas import tpu_sc as plsc

SERIES = 8192
B = 4096
L = 200
NC, NS, LN = 2, 16, 16
NW = NC * NS
RW = B // NW
G = 4
GW = G * SERIES
NG = RW // G
NB = 2
STARTS = tuple(c * LN for c in range(12)) + (L - LN,)

_mesh = plsc.VectorSubcoreMesh(core_axis_name="c", subcore_axis_name="s")


@functools.partial(
    pl.kernel,
    out_type=jax.ShapeDtypeStruct((B, SERIES), jnp.float32),
    mesh=_mesh,
    scratch_types=[
        pltpu.VMEM((RW, L), jnp.int32),
        pltpu.VMEM((NB * G, SERIES), jnp.float32),
        pltpu.SemaphoreType.DMA,
        pltpu.SemaphoreType.DMA,
    ],
    compiler_params=pltpu.CompilerParams(needs_layout_passes=False),
)
def _sc_build(x_hbm, out_hbm, idx_v, rows_v, sem0, sem1):
    wid = lax.axis_index("s") * NC + lax.axis_index("c")
    orow = wid * RW
    sems = (sem0, sem1)

    idx_dma = pltpu.async_copy(x_hbm.at[pl.ds(wid * RW, RW)], idx_v, sem1)
    zero16 = jnp.zeros((LN,), jnp.float32)

    def zero_buf(b):
        def zbody(p, _):
            for j in range(G):
                rows_v[b * G + j, pl.ds(p * LN, LN)] = zero16
            return 0

        lax.fori_loop(0, SERIES // LN, zbody, 0)

    zero_buf(0)
    idx_dma.wait()

    one16 = jnp.full((LN,), 1.0, jnp.float32)
    zero16i = jnp.zeros((LN,), jnp.int32)
    lane0 = lax.iota(jnp.int32, LN) == zero16i

    def scat(g, brow, val):
        for r in range(G):
            row = g * G + r
            rvec = jnp.full((LN,), brow + r, jnp.int32)

            @plsc.parallel_loop(0, len(STARTS), unroll=13)
            def _cbody(c):
                start = jnp.minimum(c * LN, L - LN)
                idx = idx_v[row, pl.ds(start, LN)]
                plsc.store_scatter(rows_v, [rvec, idx], val)

    def build(g, brow):
        scat(g, brow, one16)
        for r in range(G):
            rvec = jnp.full((LN,), brow + r, jnp.int32)
            plsc.store_scatter(rows_v, [rvec, zero16i], zero16, mask=lane0)

    def dma_start(b, g):
        pltpu.async_copy(
            rows_v.at[pl.ds(b * G, G)],
            out_hbm.at[pl.ds(orow + g * G, G)],
            sems[b],
        )

    def dma_wait(b):
        pltpu.make_async_copy(
            rows_v.at[pl.ds(b * G, G)],
            out_hbm.at[pl.ds(orow, G)],
            sems[b],
        ).wait()

    build(jnp.int32(0), 0)
    dma_start(0, jnp.int32(0))
    zero_buf(1)
    build(jnp.int32(1), G)
    dma_start(1, jnp.int32(1))

    def outer(t, _):
        for b in range(NB):
            g = t * NB + b
            dma_wait(b)
            scat(g - NB, b * G, zero16)
            build(g, b * G)
            dma_start(b, g)
        return 0

    lax.fori_loop(1, NG // NB, outer, 0)

    for b in range(NB):
        dma_wait(b)


def kernel(x):
    return _sc_build(x.reshape(B, L))


# speedup vs baseline: 55.0076x; 1.0072x over previous
import functools

import jax
import jax.numpy as jnp
from jax import lax
from jax.experimental import pallas as pl
from jax.experimental.pallas import tpu as pltpu
from jax.experimental.pallas import tpu_sc as plsc

SERIES = 8192
B = 4096
L = 200
NC, NS, LN = 2, 16, 16
NW = NC * NS
RW = B // NW
G = 4
GW = G * SERIES
NG = RW // G
NB = 2
STARTS = tuple(c * LN for c in range(12)) + (L - LN,)

_mesh = plsc.VectorSubcoreMesh(core_axis_name="c", subcore_axis_name="s")


@functools.partial(
    pl.kernel,
    out_type=jax.ShapeDtypeStruct((B, SERIES), jnp.float32),
    mesh=_mesh,
    scratch_types=[
        pltpu.VMEM((RW, L), jnp.int32),
        pltpu.VMEM((NB * G, SERIES), jnp.float32),
        pltpu.SemaphoreType.DMA,
        pltpu.SemaphoreType.DMA,
    ],
    compiler_params=pltpu.CompilerParams(needs_layout_passes=False),
)
def _sc_build(x_hbm, out_hbm, idx_v, rows_v, sem0, sem1):
    wid = lax.axis_index("s") * NC + lax.axis_index("c")
    orow = wid * RW
    sems = (sem0, sem1)

    idx_dma = pltpu.async_copy(x_hbm.at[pl.ds(wid * RW, RW)], idx_v, sem1)
    zero16 = jnp.zeros((LN,), jnp.float32)

    def zero_buf(b):
        @plsc.parallel_loop(0, SERIES // LN, unroll=8)
        def _zbody(p):
            for j in range(G):
                rows_v[b * G + j, pl.ds(p * LN, LN)] = zero16

    zero_buf(0)
    idx_dma.wait()

    one16 = jnp.full((LN,), 1.0, jnp.float32)
    zero16i = jnp.zeros((LN,), jnp.int32)
    lane0 = lax.iota(jnp.int32, LN) == zero16i

    def scat(g, brow, val):
        for r in range(G):
            row = g * G + r
            rvec = jnp.full((LN,), brow + r, jnp.int32)

            @plsc.parallel_loop(0, len(STARTS), unroll=13)
            def _cbody(c):
                start = jnp.minimum(c * LN, L - LN)
                idx = idx_v[row, pl.ds(start, LN)]
                plsc.store_scatter(rows_v, [rvec, idx], val)

    def build(g, brow):
        scat(g, brow, one16)
        for r in range(G):
            rvec = jnp.full((LN,), brow + r, jnp.int32)
            plsc.store_scatter(rows_v, [rvec, zero16i], zero16, mask=lane0)

    def dma_start(b, g):
        pltpu.async_copy(
            rows_v.at[pl.ds(b * G, G)],
            out_hbm.at[pl.ds(orow + g * G, G)],
            sems[b],
        )

    def dma_wait(b):
        pltpu.make_async_copy(
            rows_v.at[pl.ds(b * G, G)],
            out_hbm.at[pl.ds(orow, G)],
            sems[b],
        ).wait()

    build(jnp.int32(0), 0)
    dma_start(0, jnp.int32(0))
    zero_buf(1)
    build(jnp.int32(1), G)
    dma_start(1, jnp.int32(1))

    def outer(t, _):
        for b in range(NB):
            g = t * NB + b
            dma_wait(b)
            scat(g - NB, b * G, zero16)
            build(g, b * G)
            dma_start(b, g)
        return 0

    lax.fori_loop(1, NG // NB, outer, 0)

    for b in range(NB):
        dma_wait(b)


def kernel(x):
    return _sc_build(x.reshape(B, L))
